# Initial kernel scaffold; baseline (speedup 1.0000x reference)
import jax
import jax.numpy as jnp
from jax.experimental import pallas as pl


def kernel(x_0, t, atom_types, edge_index, bond_types, batch_idx, noise, params):
    raise NotImplementedError("write your pallas kernel here")


# speedup vs baseline: 1.0495x; 1.0495x over previous
import functools
import math

import jax
import jax.numpy as jnp
from jax.experimental import pallas as pl

HID = 128
TIME_DIM = 128
NUM_LAYERS = 6
T_STEPS = 1000


def _schedule():
    s = 0.008
    steps = T_STEPS + 1
    x = jnp.linspace(0.0, float(T_STEPS), steps)
    ac = jnp.cos((x / T_STEPS + s) / (1 + s) * math.pi * 0.5) ** 2
    ac = ac / ac[0]
    betas = jnp.clip(1 - ac[1:] / ac[:-1], 0.0001, 0.9999)
    alphas = 1 - betas
    acp = jnp.cumprod(alphas)
    return jnp.sqrt(acp), jnp.sqrt(1 - acp)


def _sin_emb(t, dim):
    half = dim // 2
    emb = jnp.exp(jnp.arange(half, dtype=jnp.float32) * -(math.log(10000.0) / (half - 1)))
    emb = t[:, None] * emb[None, :]
    return jnp.concatenate([jnp.sin(emb), jnp.cos(emb)], axis=-1)


def _silu(v):
    return v * jax.nn.sigmoid(v)


def _edge_block_kernel(ga_ref, gb_ref, bt_ref, w2t_ref, cw1t_ref, cw2p_ref,
                       wd_ref, btab_ref, eb2_ref, cb1_ref, misc_ref,
                       m_ref, cwu_ref):
    ga = ga_ref[...]
    gb = gb_ref[...]
    g = ga[:, :HID] + gb[:, :HID]
    dx = ga[:, HID:HID + 16] + gb[:, HID:HID + 16]
    d3 = dx[:, 0:3]
    d2 = jnp.sum(d3 * d3, axis=1, keepdims=True)
    dist = jnp.maximum(jnp.sqrt(d2), 1e-6)
    unit = d3 / dist
    bt = bt_ref[...]
    pre = g + dist * wd_ref[0:1, :]
    for k in range(5):
        pre += jnp.where(bt == k, btab_ref[k:k + 1, :], 0.0)
    s1 = _silu(pre)
    m = _silu(jnp.dot(s1, w2t_ref[...], preferred_element_type=jnp.float32)
              + eb2_ref[0:1, :])
    c1 = _silu(jnp.dot(m, cw1t_ref[...], preferred_element_type=jnp.float32)
               + cb1_ref[0:1, :])
    cw = jnp.sum(c1 * cw2p_ref[0:1, :], axis=1, keepdims=True) + misc_ref[0:1, 0:1]
    m_ref[...] = m
    cwu_ref[...] = jnp.concatenate(
        [cw * unit, jnp.zeros((unit.shape[0], 13), jnp.float32)], axis=1)


def _pick_blk(e):
    for b in (1280, 640, 512, 256, 128, 64, 32, 16, 8):
        if e % b == 0:
            return b
    return None


@functools.partial(jax.jit, static_argnames=())
def _edge_mlp(ga, gb, btc, w2t, cw1t, cw2p, wd, btab, eb2, cb1, misc):
    e = ga.shape[0]
    blk = _pick_blk(e)
    if blk is None:
        blk = 1280
        epad = ((e + blk - 1) // blk) * blk
        ga = jnp.pad(ga, ((0, epad - e), (0, 0)))
        gb = jnp.pad(gb, ((0, epad - e), (0, 0)))
        btc = jnp.pad(btc, ((0, epad - e), (0, 0)))
    epad = ga.shape[0]
    grid = epad // blk
    data_spec = lambda w: pl.BlockSpec((blk, w), lambda i: (i, 0))
    const_spec = lambda s: pl.BlockSpec(s, lambda i: (0, 0))
    m, cwu = pl.pallas_call(
        _edge_block_kernel,
        grid=(grid,),
        in_specs=[
            data_spec(HID + 16), data_spec(HID + 16), data_spec(1),
            const_spec((HID, HID)), const_spec((HID, 64)), const_spec((8, 64)),
            const_spec((8, HID)), const_spec((8, HID)), const_spec((8, HID)),
            const_spec((8, 64)), const_spec((8, HID)),
        ],
        out_specs=[
            pl.BlockSpec((blk, HID), lambda i: (i, 0)),
            pl.BlockSpec((blk, 16), lambda i: (i, 0)),
        ],
        out_shape=[
            jax.ShapeDtypeStruct((epad, HID), jnp.float32),
            jax.ShapeDtypeStruct((epad, 16), jnp.float32),
        ],
    )(ga, gb, btc, w2t, cw1t, cw2p, wd, btab, eb2, cb1, misc)
    return m[:e], cwu[:e]


def _pad8(v, rows=8):
    v = jnp.reshape(v, (-1,))
    out = jnp.zeros((rows, v.shape[0]), jnp.float32)
    return out.at[0, :].set(v)


def kernel(x_0, t, atom_types, edge_index, bond_types, batch_idx, noise, params):
    p = params
    sqa, sqm = _schedule()
    sa = sqa[t][batch_idx][:, None]
    sm = sqm[t][batch_idx][:, None]
    x_t = sa * x_0 + sm * noise

    h = p['atom_embed'][jnp.clip(atom_types, 0, 9)]
    h = h + x_t @ p['coord_w'].T + p['coord_b']
    te = _sin_emb(t.astype(jnp.float32), TIME_DIM)
    te = _silu(te @ p['time_w1'].T + p['time_b1']) @ p['time_w2'].T + p['time_b2']
    h = h + te[batch_idx]

    row, col = edge_index[0], edge_index[1]
    btc = jnp.clip(bond_types, 0, 4).astype(jnp.int32)[:, None]
    n = x_0.shape[0]
    x = x_t

    for lp in p['layers']:
        w_hr = lp['e_w1'][:, :HID]
        w_hc = lp['e_w1'][:, HID:2 * HID]
        w_ea = lp['e_w1'][:, 2 * HID:2 * HID + 32]
        w_d = lp['e_w1'][:, 2 * HID + 32]
        a_tab = h @ w_hr.T + lp['e_b1']
        b_tab = h @ w_hc.T
        zpad = jnp.zeros((n, 13), jnp.float32)
        ta = jnp.concatenate([a_tab, x, zpad], axis=1)
        tb = jnp.concatenate([b_tab, -x, zpad], axis=1)
        ga = jnp.take(ta, row, axis=0)
        gb = jnp.take(tb, col, axis=0)

        btab6 = p['bond_embed'] @ w_ea.T
        btab = jnp.zeros((8, HID), jnp.float32).at[:6].set(btab6)
        misc = jnp.zeros((8, HID), jnp.float32).at[0, 0].set(lp['c_b2'][0])
        m, cwu = _edge_mlp(
            ga, gb, btc,
            lp['e_w2'].T, lp['c_w1'].T, _pad8(lp['c_w2'][0][None, :]),
            _pad8(w_d[None, :]), btab, _pad8(lp['e_b2'][None, :]),
            _pad8(lp['c_b1'][None, :]), misc)

        x = x + jnp.zeros_like(x).at[col].add(cwu[:, :3])
        m_agg = jnp.zeros_like(h).at[col].add(m)
        nin = jnp.concatenate([h, m_agg], axis=1)
        h_new = _silu(nin @ lp['n_w1'].T + lp['n_b1']) @ lp['n_w2'].T + lp['n_b2']
        hr = h + h_new
        mu = hr.mean(-1, keepdims=True)
        var = hr.var(-1, keepdims=True)
        h = (hr - mu) / jnp.sqrt(var + 1e-05) * lp['ln_g'] + lp['ln_b']

    return _silu(h @ p['np_w1'].T + p['np_b1']) @ p['np_w2'].T + p['np_b2']


# speedup vs baseline: 1.4107x; 1.3442x over previous
import functools
import math

import jax
import jax.numpy as jnp
from jax import lax
from jax.experimental import pallas as pl
from jax.experimental.pallas import tpu as pltpu
from jax.experimental.pallas import tpu_sc as plsc

HID = 128
TIME_DIM = 128
NUM_LAYERS = 6
T_STEPS = 1000


def _schedule():
    s = 0.008
    steps = T_STEPS + 1
    x = jnp.linspace(0.0, float(T_STEPS), steps)
    ac = jnp.cos((x / T_STEPS + s) / (1 + s) * math.pi * 0.5) ** 2
    ac = ac / ac[0]
    betas = jnp.clip(1 - ac[1:] / ac[:-1], 0.0001, 0.9999)
    alphas = 1 - betas
    acp = jnp.cumprod(alphas)
    return jnp.sqrt(acp), jnp.sqrt(1 - acp)


def _sin_emb(t, dim):
    half = dim // 2
    emb = jnp.exp(jnp.arange(half, dtype=jnp.float32) * -(math.log(10000.0) / (half - 1)))
    emb = t[:, None] * emb[None, :]
    return jnp.concatenate([jnp.sin(emb), jnp.cos(emb)], axis=-1)


def _silu(v):
    return v * jax.nn.sigmoid(v)


def _edge_block_kernel(ga_ref, gb_ref, dx_ref, bt_ref, w2t_ref, cw1t_ref,
                       cw2p_ref, wd_ref, btab_ref, eb2_ref, cb1_ref, misc_ref,
                       m_ref, cwu_ref):
    g = ga_ref[...] + gb_ref[...]
    dx = dx_ref[...]
    d3 = dx[:, 0:3]
    d2 = jnp.sum(d3 * d3, axis=1, keepdims=True)
    dist = jnp.maximum(jnp.sqrt(d2), 1e-6)
    unit = d3 / dist
    bt = bt_ref[...]
    pre = g + dist * wd_ref[0:1, :]
    for k in range(5):
        pre += jnp.where(bt == k, btab_ref[k:k + 1, :], 0.0)
    s1 = _silu(pre)
    m = _silu(jnp.dot(s1, w2t_ref[...], preferred_element_type=jnp.float32)
              + eb2_ref[0:1, :])
    c1 = _silu(jnp.dot(m, cw1t_ref[...], preferred_element_type=jnp.float32)
               + cb1_ref[0:1, :])
    cw = jnp.sum(c1 * cw2p_ref[0:1, :], axis=1, keepdims=True) + misc_ref[0:1, 0:1]
    m_ref[...] = m
    cwu_ref[...] = jnp.concatenate(
        [cw * unit, jnp.zeros((unit.shape[0], 13), jnp.float32)], axis=1)


def _node_block_kernel(h_ref, magg_ref, w1at_ref, w1bt_ref, w2t_ref,
                       nb1_ref, nb2_ref, lng_ref, lnb_ref, out_ref):
    h = h_ref[...]
    magg = magg_ref[...]
    z = (jnp.dot(h, w1at_ref[...], preferred_element_type=jnp.float32)
         + jnp.dot(magg, w1bt_ref[...], preferred_element_type=jnp.float32)
         + nb1_ref[0:1, :])
    hn = jnp.dot(_silu(z), w2t_ref[...],
                 preferred_element_type=jnp.float32) + nb2_ref[0:1, :]
    hr = h + hn
    mu = jnp.mean(hr, axis=1, keepdims=True)
    var = jnp.mean((hr - mu) * (hr - mu), axis=1, keepdims=True)
    out_ref[...] = ((hr - mu) * lax.rsqrt(var + 1e-05) * lng_ref[0:1, :]
                    + lnb_ref[0:1, :])


def _prep_block_kernel(h_ref, whrt_ref, whct_ref, eb1_ref, ta_ref, tb_ref):
    h = h_ref[...]
    ta_ref[...] = jnp.dot(h, whrt_ref[...],
                          preferred_element_type=jnp.float32) + eb1_ref[0:1, :]
    tb_ref[...] = jnp.dot(h, whct_ref[...],
                          preferred_element_type=jnp.float32)


def _head_block_kernel(h_ref, w1t_ref, b1_ref, w2tp_ref, b2_ref, out_ref):
    z = _silu(jnp.dot(h_ref[...], w1t_ref[...],
                      preferred_element_type=jnp.float32) + b1_ref[0:1, :])
    out_ref[...] = jnp.dot(z, w2tp_ref[...],
                           preferred_element_type=jnp.float32) + b2_ref[0:1, :]


def _node_blk(n):
    for b in (2000, 1000, 500, 250, 200, 100, 50, 25, 8):
        if n % b == 0 and b % 8 == 0:
            return b
    return None


def _node_update(h, magg, w1at, w1bt, w2t, nb1, nb2, lng, lnb):
    n = h.shape[0]
    blk = _node_blk(n)
    if blk is None:
        z = h @ w1at + magg @ w1bt + nb1[0:1]
        hr = h + (_silu(z) @ w2t + nb2[0:1])
        mu = hr.mean(-1, keepdims=True)
        var = hr.var(-1, keepdims=True)
        return (hr - mu) / jnp.sqrt(var + 1e-05) * lng[0:1] + lnb[0:1]
    data = pl.BlockSpec((blk, HID), lambda i: (i, 0))
    const = lambda s: pl.BlockSpec(s, lambda i: (0, 0))
    return pl.pallas_call(
        _node_block_kernel,
        grid=(n // blk,),
        in_specs=[data, data, const((HID, HID)), const((HID, HID)),
                  const((HID, HID)), const((8, HID)), const((8, HID)),
                  const((8, HID)), const((8, HID))],
        out_specs=data,
        out_shape=jax.ShapeDtypeStruct((n, HID), jnp.float32),
    )(h, magg, w1at, w1bt, w2t, nb1, nb2, lng, lnb)


def _prep_tables(h, whrt, whct, eb1):
    n = h.shape[0]
    blk = _node_blk(n)
    if blk is None:
        return h @ whrt + eb1[0:1], h @ whct
    data = pl.BlockSpec((blk, HID), lambda i: (i, 0))
    const = lambda s: pl.BlockSpec(s, lambda i: (0, 0))
    return pl.pallas_call(
        _prep_block_kernel,
        grid=(n // blk,),
        in_specs=[data, const((HID, HID)), const((HID, HID)), const((8, HID))],
        out_specs=[data, data],
        out_shape=[jax.ShapeDtypeStruct((n, HID), jnp.float32),
                   jax.ShapeDtypeStruct((n, HID), jnp.float32)],
    )(h, whrt, whct, eb1)


def _head(h, w1t, b1, w2tp, b2p):
    n = h.shape[0]
    blk = _node_blk(n)
    if blk is None:
        return (_silu(h @ w1t + b1[0:1]) @ w2tp + b2p[0:1])[:, :3]
    data = pl.BlockSpec((blk, HID), lambda i: (i, 0))
    const = lambda s: pl.BlockSpec(s, lambda i: (0, 0))
    out = pl.pallas_call(
        _head_block_kernel,
        grid=(n // blk,),
        in_specs=[data, const((HID, HID)), const((8, HID)),
                  const((HID, 8)), const((8, 8))],
        out_specs=pl.BlockSpec((blk, 8), lambda i: (i, 0)),
        out_shape=jax.ShapeDtypeStruct((n, 8), jnp.float32),
    )(h, w1t, b1, w2tp, b2p)
    return out[:, :3]


def _pick_blk(e):
    for b in (1280, 640, 512, 256, 128, 64, 32, 16, 8):
        if e % b == 0:
            return b
    return None


def _edge_mlp(ga, gb, dxp, btc, w2t, cw1t, cw2p, wd, btab, eb2, cb1, misc):
    e = ga.shape[0]
    blk = _pick_blk(e)
    if blk is None:
        blk = 1280
        epad = ((e + blk - 1) // blk) * blk
        ga = jnp.pad(ga, ((0, epad - e), (0, 0)))
        gb = jnp.pad(gb, ((0, epad - e), (0, 0)))
        dxp = jnp.pad(dxp, ((0, epad - e), (0, 0)))
        btc = jnp.pad(btc, ((0, epad - e), (0, 0)))
    epad = ga.shape[0]
    grid = epad // blk
    data_spec = lambda w: pl.BlockSpec((blk, w), lambda i: (i, 0))
    const_spec = lambda s: pl.BlockSpec(s, lambda i: (0, 0))
    m, cwu = pl.pallas_call(
        _edge_block_kernel,
        grid=(grid,),
        in_specs=[
            data_spec(HID), data_spec(HID), data_spec(16), data_spec(1),
            const_spec((HID, HID)), const_spec((HID, 64)), const_spec((8, 64)),
            const_spec((8, HID)), const_spec((8, HID)), const_spec((8, HID)),
            const_spec((8, 64)), const_spec((8, HID)),
        ],
        out_specs=[
            pl.BlockSpec((blk, HID), lambda i: (i, 0)),
            pl.BlockSpec((blk, 16), lambda i: (i, 0)),
        ],
        out_shape=[
            jax.ShapeDtypeStruct((epad, HID), jnp.float32),
            jax.ShapeDtypeStruct((epad, 16), jnp.float32),
        ],
    )(ga, gb, dxp, btc, w2t, cw1t, cw2p, wd, btab, eb2, cb1, misc)
    return m[:e], cwu[:e]


_NC, _NS = 2, 16
_NW = _NC * _NS


def _gather_chunk(epw):
    for k in range(128, 7, -8):
        if epw % k == 0:
            return k
    return None


@functools.lru_cache(maxsize=None)
def _sc_gather_kernel(n, w, e):
    epw = e // _NW
    k_chunk = _gather_chunk(epw)
    mesh = plsc.VectorSubcoreMesh(core_axis_name="c", subcore_axis_name="s")

    @functools.partial(
        pl.kernel, mesh=mesh,
        out_type=[jax.ShapeDtypeStruct((e, w), jnp.float32),
                  jax.ShapeDtypeStruct((e, w), jnp.float32)],
        scratch_types=[
            pltpu.VMEM((k_chunk,), jnp.int32),
            pltpu.VMEM((k_chunk,), jnp.int32),
            pltpu.VMEM((k_chunk, w), jnp.float32),
            pltpu.VMEM((k_chunk, w), jnp.float32),
            pltpu.SemaphoreType.DMA,
            pltpu.SemaphoreType.DMA,
        ],
    )
    def gather_kernel(ta_h, tb_h, row_h, col_h, ga_h, gb_h,
                      idx_r, idx_c, buf_a, buf_b, sem_a, sem_b):
        wid = lax.axis_index("s") * _NC + lax.axis_index("c")
        base = wid * epw

        def body(j, carry):
            off = base + j * k_chunk
            pltpu.sync_copy(row_h.at[pl.ds(off, k_chunk)], idx_r)
            pltpu.sync_copy(col_h.at[pl.ds(off, k_chunk)], idx_c)
            cp_a = pltpu.async_copy(ta_h.at[idx_r], buf_a, sem_a)
            cp_b = pltpu.async_copy(tb_h.at[idx_c], buf_b, sem_b)
            cp_a.wait()
            cp_b.wait()
            pltpu.sync_copy(buf_a, ga_h.at[pl.ds(off, k_chunk)])
            pltpu.sync_copy(buf_b, gb_h.at[pl.ds(off, k_chunk)])
            return carry

        lax.fori_loop(0, epw // k_chunk, body, 0)

    return gather_kernel


def _sc_gather(ta, tb, row, col):
    return _sc_gather_kernel(ta.shape[0], ta.shape[1], row.shape[0])(
        ta, tb, row, col)


@functools.lru_cache(maxsize=None)
def _sc_scatter_kernel(e, n):
    epw = e // _NW
    k_chunk = _gather_chunk(epw)
    n_chunks = n // k_chunk
    mesh = plsc.VectorSubcoreMesh(core_axis_name="c", subcore_axis_name="s")
    stripe_iters = (n_chunks + _NS - 1) // _NS

    @functools.partial(
        pl.kernel, mesh=mesh,
        out_type=jax.ShapeDtypeStruct((2 * n, HID), jnp.float32),
        scratch_types=[
            pltpu.VMEM((k_chunk,), jnp.int32),
            pltpu.VMEM((k_chunk, HID), jnp.float32),
            pltpu.VMEM((k_chunk, HID), jnp.float32),
            pltpu.VMEM_SHARED((n, HID), jnp.float32),
        ],
    )
    def scatter_kernel(m_h, col_h, z_h, out_h, idx_v, m_v, z_v, acc_sh):
        cid = lax.axis_index("c")
        sid = lax.axis_index("s")
        wid = sid * _NC + cid
        base = wid * epw

        pltpu.sync_copy(z_h, z_v)

        def zero_body(j, carry):
            chunk = j * _NS + sid

            @pl.when(chunk < n_chunks)
            def _():
                pltpu.sync_copy(z_v, acc_sh.at[pl.ds(chunk * k_chunk, k_chunk)])
            return carry

        lax.fori_loop(0, stripe_iters, zero_body, 0)
        plsc.subcore_barrier()

        def body(j, carry):
            off = base + j * k_chunk
            pltpu.sync_copy(col_h.at[pl.ds(off, k_chunk)], idx_v)
            pltpu.sync_copy(m_h.at[pl.ds(off, k_chunk)], m_v)
            pltpu.sync_copy(m_v, acc_sh.at[idx_v], add=True)
            return carry

        lax.fori_loop(0, epw // k_chunk, body, 0)
        plsc.subcore_barrier()

        def out_body(j, carry):
            chunk = j * _NS + sid

            @pl.when(chunk < n_chunks)
            def _():
                pltpu.sync_copy(acc_sh.at[pl.ds(chunk * k_chunk, k_chunk)],
                                out_h.at[pl.ds(cid * n + chunk * k_chunk, k_chunk)])
            return carry

        lax.fori_loop(0, stripe_iters, out_body, 0)

    return scatter_kernel


def _sc_scatter(m, col, n):
    e = m.shape[0]
    k_chunk = _gather_chunk(e // _NW)
    zeros = jnp.zeros((k_chunk, HID), jnp.float32)
    return _sc_scatter_kernel(e, n)(m, col, zeros)


def _pad8(v, rows=8):
    v = jnp.reshape(v, (-1,))
    out = jnp.zeros((rows, v.shape[0]), jnp.float32)
    return out.at[0, :].set(v)


def kernel(x_0, t, atom_types, edge_index, bond_types, batch_idx, noise, params):
    p = params
    sqa, sqm = _schedule()
    sa = sqa[t][batch_idx][:, None]
    sm = sqm[t][batch_idx][:, None]
    x_t = sa * x_0 + sm * noise

    h = p['atom_embed'][jnp.clip(atom_types, 0, 9)]
    h = h + x_t @ p['coord_w'].T + p['coord_b']
    te = _sin_emb(t.astype(jnp.float32), TIME_DIM)
    te = _silu(te @ p['time_w1'].T + p['time_b1']) @ p['time_w2'].T + p['time_b2']
    h = h + te[batch_idx]

    row, col = edge_index[0], edge_index[1]
    btc = jnp.clip(bond_types, 0, 4).astype(jnp.int32)[:, None]
    n = x_0.shape[0]
    x = x_t
    e = row.shape[0]
    _kc = _gather_chunk(e // _NW) if e % _NW == 0 else None
    use_sc = _kc is not None and n % _kc == 0
    row_i = row.astype(jnp.int32)
    col_i = col.astype(jnp.int32)

    for lp in p['layers']:
        w_hr = lp['e_w1'][:, :HID]
        w_hc = lp['e_w1'][:, HID:2 * HID]
        w_ea = lp['e_w1'][:, 2 * HID:2 * HID + 32]
        w_d = lp['e_w1'][:, 2 * HID + 32]
        ta, tb = _prep_tables(h, w_hr.T, w_hc.T, _pad8(lp['e_b1']))
        if use_sc:
            ga, gb = _sc_gather(ta, tb, row_i, col_i)
        else:
            ga = jnp.take(ta, row, axis=0)
            gb = jnp.take(tb, col, axis=0)
        dx3 = jnp.take(x, row, axis=0) - jnp.take(x, col, axis=0)
        dxp = jnp.pad(dx3, ((0, 0), (0, 13)))

        btab6 = p['bond_embed'] @ w_ea.T
        btab = jnp.zeros((8, HID), jnp.float32).at[:6].set(btab6)
        misc = jnp.zeros((8, HID), jnp.float32).at[0, 0].set(lp['c_b2'][0])
        m, cwu = _edge_mlp(
            ga, gb, dxp, btc,
            lp['e_w2'].T, lp['c_w1'].T, _pad8(lp['c_w2'][0][None, :]),
            _pad8(w_d[None, :]), btab, _pad8(lp['e_b2'][None, :]),
            _pad8(lp['c_b1'][None, :]), misc)

        x = x + jnp.zeros_like(x).at[col].add(cwu[:, :3])
        if use_sc:
            parts = _sc_scatter(m, col_i, n)
            m_agg = parts[:n] + parts[n:]
        else:
            m_agg = jnp.zeros_like(h).at[col].add(m)
        h = _node_update(
            h, m_agg, lp['n_w1'][:, :HID].T, lp['n_w1'][:, HID:].T,
            lp['n_w2'].T, _pad8(lp['n_b1']), _pad8(lp['n_b2']),
            _pad8(lp['ln_g']), _pad8(lp['ln_b']))

    w2tp = jnp.zeros((HID, 8), jnp.float32).at[:, :3].set(p['np_w2'].T)
    b2p = jnp.zeros((8, 8), jnp.float32).at[0, :3].set(p['np_b2'])
    return _head(h, p['np_w1'].T, _pad8(p['np_b1']), w2tp, b2p)


# speedup vs baseline: 1.5461x; 1.0960x over previous
import functools
import math

import jax
import jax.numpy as jnp
from jax import lax
from jax.experimental import pallas as pl
from jax.experimental.pallas import tpu as pltpu
from jax.experimental.pallas import tpu_sc as plsc

HID = 128
TIME_DIM = 128
NUM_LAYERS = 6
T_STEPS = 1000


def _schedule():
    s = 0.008
    steps = T_STEPS + 1
    x = jnp.linspace(0.0, float(T_STEPS), steps)
    ac = jnp.cos((x / T_STEPS + s) / (1 + s) * math.pi * 0.5) ** 2
    ac = ac / ac[0]
    betas = jnp.clip(1 - ac[1:] / ac[:-1], 0.0001, 0.9999)
    alphas = 1 - betas
    acp = jnp.cumprod(alphas)
    return jnp.sqrt(acp), jnp.sqrt(1 - acp)


def _sin_emb(t, dim):
    half = dim // 2
    emb = jnp.exp(jnp.arange(half, dtype=jnp.float32) * -(math.log(10000.0) / (half - 1)))
    emb = t[:, None] * emb[None, :]
    return jnp.concatenate([jnp.sin(emb), jnp.cos(emb)], axis=-1)


def _silu(v):
    return v * jax.nn.sigmoid(v)


def _edge_block_kernel(ga_ref, gb_ref, dx_ref, bt_ref, w2t_ref, cw1t_ref,
                       cw2p_ref, wd_ref, btab_ref, eb2_ref, cb1_ref, misc_ref,
                       m_ref, cwu_ref):
    g = ga_ref[...] + gb_ref[...]
    dx = dx_ref[...]
    d3 = dx[:, 0:3]
    d2 = jnp.sum(d3 * d3, axis=1, keepdims=True)
    dist = jnp.maximum(jnp.sqrt(d2), 1e-6)
    unit = d3 / dist
    bt = bt_ref[...]
    pre = g + dist * wd_ref[0:1, :]
    for k in range(5):
        pre += jnp.where(bt == k, btab_ref[k:k + 1, :], 0.0)
    s1 = _silu(pre)
    m = _silu(jnp.dot(s1, w2t_ref[...], preferred_element_type=jnp.float32)
              + eb2_ref[0:1, :])
    c1 = _silu(jnp.dot(m, cw1t_ref[...], preferred_element_type=jnp.float32)
               + cb1_ref[0:1, :])
    cw = jnp.sum(c1 * cw2p_ref[0:1, :], axis=1, keepdims=True) + misc_ref[0:1, 0:1]
    m_ref[...] = m
    cwu_ref[...] = jnp.concatenate(
        [cw * unit, jnp.zeros((unit.shape[0], HID - 3), jnp.float32)], axis=1)


def _node_block_kernel(h_ref, magg_ref, w1at_ref, w1bt_ref, w2t_ref,
                       nb1_ref, nb2_ref, lng_ref, lnb_ref, out_ref):
    h = h_ref[...]
    magg = magg_ref[...]
    z = (jnp.dot(h, w1at_ref[...], preferred_element_type=jnp.float32)
         + jnp.dot(magg, w1bt_ref[...], preferred_element_type=jnp.float32)
         + nb1_ref[0:1, :])
    hn = jnp.dot(_silu(z), w2t_ref[...],
                 preferred_element_type=jnp.float32) + nb2_ref[0:1, :]
    hr = h + hn
    mu = jnp.mean(hr, axis=1, keepdims=True)
    var = jnp.mean((hr - mu) * (hr - mu), axis=1, keepdims=True)
    out_ref[...] = ((hr - mu) * lax.rsqrt(var + 1e-05) * lng_ref[0:1, :]
                    + lnb_ref[0:1, :])


def _prep_block_kernel(h_ref, whrt_ref, whct_ref, eb1_ref, ta_ref, tb_ref):
    h = h_ref[...]
    ta_ref[...] = jnp.dot(h, whrt_ref[...],
                          preferred_element_type=jnp.float32) + eb1_ref[0:1, :]
    tb_ref[...] = jnp.dot(h, whct_ref[...],
                          preferred_element_type=jnp.float32)


def _head_block_kernel(h_ref, w1t_ref, b1_ref, w2tp_ref, b2_ref, out_ref):
    z = _silu(jnp.dot(h_ref[...], w1t_ref[...],
                      preferred_element_type=jnp.float32) + b1_ref[0:1, :])
    out_ref[...] = jnp.dot(z, w2tp_ref[...],
                           preferred_element_type=jnp.float32) + b2_ref[0:1, :]


def _node_blk(n):
    for b in (2000, 1000, 500, 250, 200, 100, 50, 25, 8):
        if n % b == 0 and b % 8 == 0:
            return b
    return None


def _node_update(h, magg, w1at, w1bt, w2t, nb1, nb2, lng, lnb):
    n = h.shape[0]
    blk = _node_blk(n)
    if blk is None:
        z = h @ w1at + magg @ w1bt + nb1[0:1]
        hr = h + (_silu(z) @ w2t + nb2[0:1])
        mu = hr.mean(-1, keepdims=True)
        var = hr.var(-1, keepdims=True)
        return (hr - mu) / jnp.sqrt(var + 1e-05) * lng[0:1] + lnb[0:1]
    data = pl.BlockSpec((blk, HID), lambda i: (i, 0))
    const = lambda s: pl.BlockSpec(s, lambda i: (0, 0))
    return pl.pallas_call(
        _node_block_kernel,
        grid=(n // blk,),
        in_specs=[data, data, const((HID, HID)), const((HID, HID)),
                  const((HID, HID)), const((8, HID)), const((8, HID)),
                  const((8, HID)), const((8, HID))],
        out_specs=data,
        out_shape=jax.ShapeDtypeStruct((n, HID), jnp.float32),
    )(h, magg, w1at, w1bt, w2t, nb1, nb2, lng, lnb)


def _prep_tables(h, whrt, whct, eb1):
    n = h.shape[0]
    blk = _node_blk(n)
    if blk is None:
        return h @ whrt + eb1[0:1], h @ whct
    data = pl.BlockSpec((blk, HID), lambda i: (i, 0))
    const = lambda s: pl.BlockSpec(s, lambda i: (0, 0))
    return pl.pallas_call(
        _prep_block_kernel,
        grid=(n // blk,),
        in_specs=[data, const((HID, HID)), const((HID, HID)), const((8, HID))],
        out_specs=[data, data],
        out_shape=[jax.ShapeDtypeStruct((n, HID), jnp.float32),
                   jax.ShapeDtypeStruct((n, HID), jnp.float32)],
    )(h, whrt, whct, eb1)


def _head(h, w1t, b1, w2tp, b2p):
    n = h.shape[0]
    blk = _node_blk(n)
    if blk is None:
        return (_silu(h @ w1t + b1[0:1]) @ w2tp + b2p[0:1])[:, :3]
    data = pl.BlockSpec((blk, HID), lambda i: (i, 0))
    const = lambda s: pl.BlockSpec(s, lambda i: (0, 0))
    out = pl.pallas_call(
        _head_block_kernel,
        grid=(n // blk,),
        in_specs=[data, const((HID, HID)), const((8, HID)),
                  const((HID, 8)), const((8, 8))],
        out_specs=pl.BlockSpec((blk, 8), lambda i: (i, 0)),
        out_shape=jax.ShapeDtypeStruct((n, 8), jnp.float32),
    )(h, w1t, b1, w2tp, b2p)
    return out[:, :3]


def _pick_blk(e):
    for b in (1280, 640, 512, 256, 128, 64, 32, 16, 8):
        if e % b == 0:
            return b
    return None


def _edge_mlp(ga, gb, dxp, btc, w2t, cw1t, cw2p, wd, btab, eb2, cb1, misc):
    e = ga.shape[0]
    blk = _pick_blk(e)
    if blk is None:
        blk = 1280
        epad = ((e + blk - 1) // blk) * blk
        ga = jnp.pad(ga, ((0, epad - e), (0, 0)))
        gb = jnp.pad(gb, ((0, epad - e), (0, 0)))
        dxp = jnp.pad(dxp, ((0, epad - e), (0, 0)))
        btc = jnp.pad(btc, ((0, epad - e), (0, 0)))
    epad = ga.shape[0]
    grid = epad // blk
    data_spec = lambda w: pl.BlockSpec((blk, w), lambda i: (i, 0))
    const_spec = lambda s: pl.BlockSpec(s, lambda i: (0, 0))
    m, cwu = pl.pallas_call(
        _edge_block_kernel,
        grid=(grid,),
        in_specs=[
            data_spec(HID), data_spec(HID), data_spec(16), data_spec(1),
            const_spec((HID, HID)), const_spec((HID, 64)), const_spec((8, 64)),
            const_spec((8, HID)), const_spec((8, HID)), const_spec((8, HID)),
            const_spec((8, 64)), const_spec((8, HID)),
        ],
        out_specs=[
            pl.BlockSpec((blk, HID), lambda i: (i, 0)),
            pl.BlockSpec((blk, HID), lambda i: (i, 0)),
        ],
        out_shape=[
            jax.ShapeDtypeStruct((epad, HID), jnp.float32),
            jax.ShapeDtypeStruct((epad, HID), jnp.float32),
        ],
    )(ga, gb, dxp, btc, w2t, cw1t, cw2p, wd, btab, eb2, cb1, misc)
    return m[:e], cwu[:e]


_NC, _NS = 2, 16
_NW = _NC * _NS


def _gather_chunk(epw):
    for k in range(128, 7, -8):
        if epw % k == 0:
            return k
    return None


@functools.lru_cache(maxsize=None)
def _sc_gather_kernel(n, w, e):
    epw = e // _NW
    k_chunk = _gather_chunk(epw)
    n_chunks = epw // k_chunk
    grp = 1
    for g in (5, 4, 3, 2):
        if n_chunks % g == 0:
            grp = g
            break
    mesh = plsc.VectorSubcoreMesh(core_axis_name="c", subcore_axis_name="s")

    @functools.partial(
        pl.kernel, mesh=mesh,
        out_type=[jax.ShapeDtypeStruct((e, w), jnp.float32),
                  jax.ShapeDtypeStruct((e, w), jnp.float32)],
        scratch_types=[
            pltpu.VMEM((grp, k_chunk), jnp.int32),
            pltpu.VMEM((grp, k_chunk), jnp.int32),
            pltpu.VMEM((grp, k_chunk, w), jnp.float32),
            pltpu.VMEM((grp, k_chunk, w), jnp.float32),
            pltpu.SemaphoreType.DMA,
            pltpu.SemaphoreType.DMA,
        ],
    )
    def gather_kernel(ta_h, tb_h, row_h, col_h, ga_h, gb_h,
                      idx_r, idx_c, buf_a, buf_b, sem_g, sem_w):
        wid = lax.axis_index("s") * _NC + lax.axis_index("c")
        base = wid * epw

        def body(j, carry):
            goff = base + j * (grp * k_chunk)
            for b in range(grp):
                off = goff + b * k_chunk
                pltpu.sync_copy(row_h.at[pl.ds(off, k_chunk)], idx_r.at[b])
                pltpu.sync_copy(col_h.at[pl.ds(off, k_chunk)], idx_c.at[b])
            cps = []
            for b in range(grp):
                cps.append(pltpu.async_copy(ta_h.at[idx_r.at[b]],
                                            buf_a.at[b], sem_g))
                cps.append(pltpu.async_copy(tb_h.at[idx_c.at[b]],
                                            buf_b.at[b], sem_g))
            for cp in cps:
                cp.wait()
            wps = []
            for b in range(grp):
                off = goff + b * k_chunk
                wps.append(pltpu.async_copy(
                    buf_a.at[b], ga_h.at[pl.ds(off, k_chunk)], sem_w))
                wps.append(pltpu.async_copy(
                    buf_b.at[b], gb_h.at[pl.ds(off, k_chunk)], sem_w))
            for wp in wps:
                wp.wait()
            return carry

        lax.fori_loop(0, n_chunks // grp, body, 0)

    return gather_kernel


def _sc_gather(ta, tb, row, col):
    return _sc_gather_kernel(ta.shape[0], ta.shape[1], row.shape[0])(
        ta, tb, row, col)


@functools.lru_cache(maxsize=None)
def _sc_scatter_kernel(e, n):
    epw = e // _NW
    k_chunk = _scatter_chunk(epw, n)
    n_chunks = n // k_chunk
    e_chunks = epw // k_chunk
    grp = 1
    for g in (5, 4, 3, 2):
        if e_chunks % g == 0:
            grp = g
            break
    mesh = plsc.VectorSubcoreMesh(core_axis_name="c", subcore_axis_name="s")
    stripe_iters = (n_chunks + _NS - 1) // _NS

    @functools.partial(
        pl.kernel, mesh=mesh,
        out_type=jax.ShapeDtypeStruct((2 * n, HID), jnp.float32),
        scratch_types=[
            pltpu.VMEM((grp, k_chunk), jnp.int32),
            pltpu.VMEM((grp, k_chunk, HID), jnp.float32),
            pltpu.VMEM((k_chunk, HID), jnp.float32),
            pltpu.VMEM_SHARED((n, HID), jnp.float32),
            pltpu.SemaphoreType.DMA,
        ],
    )
    def scatter_kernel(m_h, col_h, z_h, out_h, idx_v, m_v, z_v, acc_sh, sem_l):
        cid = lax.axis_index("c")
        sid = lax.axis_index("s")
        wid = sid * _NC + cid
        base = wid * epw

        pltpu.sync_copy(z_h, z_v)

        def zero_body(j, carry):
            chunk = j * _NS + sid

            @pl.when(chunk < n_chunks)
            def _():
                pltpu.sync_copy(z_v, acc_sh.at[pl.ds(chunk * k_chunk, k_chunk)])
            return carry

        lax.fori_loop(0, stripe_iters, zero_body, 0)
        plsc.subcore_barrier()

        def body(j, carry):
            goff = base + j * (grp * k_chunk)
            cps = []
            for b in range(grp):
                off = goff + b * k_chunk
                pltpu.sync_copy(col_h.at[pl.ds(off, k_chunk)], idx_v.at[b])
                cps.append(pltpu.async_copy(m_h.at[pl.ds(off, k_chunk)],
                                            m_v.at[b], sem_l))
            for cp in cps:
                cp.wait()
            for b in range(grp):
                pltpu.sync_copy(m_v.at[b], acc_sh.at[idx_v.at[b]], add=True)
            return carry

        lax.fori_loop(0, e_chunks // grp, body, 0)
        plsc.subcore_barrier()

        def out_body(j, carry):
            chunk = j * _NS + sid

            @pl.when(chunk < n_chunks)
            def _():
                pltpu.sync_copy(acc_sh.at[pl.ds(chunk * k_chunk, k_chunk)],
                                out_h.at[pl.ds(cid * n + chunk * k_chunk, k_chunk)])
            return carry

        lax.fori_loop(0, stripe_iters, out_body, 0)

    return scatter_kernel


def _scatter_chunk(epw, n):
    for k in range(64, 7, -8):
        if epw % k == 0 and n % k == 0:
            return k
    return None


def _sc_scatter(m, col, n):
    e = m.shape[0]
    k_chunk = _scatter_chunk(e // _NW, n)
    zeros = jnp.zeros((k_chunk, HID), jnp.float32)
    return _sc_scatter_kernel(e, n)(m, col, zeros)


def _pad8(v, rows=8):
    v = jnp.reshape(v, (-1,))
    out = jnp.zeros((rows, v.shape[0]), jnp.float32)
    return out.at[0, :].set(v)


def kernel(x_0, t, atom_types, edge_index, bond_types, batch_idx, noise, params):
    p = params
    sqa, sqm = _schedule()
    sa = sqa[t][batch_idx][:, None]
    sm = sqm[t][batch_idx][:, None]
    x_t = sa * x_0 + sm * noise

    h = p['atom_embed'][jnp.clip(atom_types, 0, 9)]
    h = h + x_t @ p['coord_w'].T + p['coord_b']
    te = _sin_emb(t.astype(jnp.float32), TIME_DIM)
    te = _silu(te @ p['time_w1'].T + p['time_b1']) @ p['time_w2'].T + p['time_b2']
    h = h + te[batch_idx]

    row, col = edge_index[0], edge_index[1]
    btc = jnp.clip(bond_types, 0, 4).astype(jnp.int32)[:, None]
    n = x_0.shape[0]
    x = x_t
    e = row.shape[0]
    use_sc = (e % _NW == 0 and _gather_chunk(e // _NW) is not None
              and _scatter_chunk(e // _NW, n) is not None)
    row_i = row.astype(jnp.int32)
    col_i = col.astype(jnp.int32)

    for li, lp in enumerate(p['layers']):
        w_hr = lp['e_w1'][:, :HID]
        w_hc = lp['e_w1'][:, HID:2 * HID]
        w_ea = lp['e_w1'][:, 2 * HID:2 * HID + 32]
        w_d = lp['e_w1'][:, 2 * HID + 32]
        ta, tb = _prep_tables(h, w_hr.T, w_hc.T, _pad8(lp['e_b1']))
        if use_sc:
            ga, gb = _sc_gather(ta, tb, row_i, col_i)
        else:
            ga = jnp.take(ta, row, axis=0)
            gb = jnp.take(tb, col, axis=0)
        dx3 = jnp.take(x, row, axis=0) - jnp.take(x, col, axis=0)
        dxp = jnp.pad(dx3, ((0, 0), (0, 13)))

        btab6 = p['bond_embed'] @ w_ea.T
        btab = jnp.zeros((8, HID), jnp.float32).at[:6].set(btab6)
        misc = jnp.zeros((8, HID), jnp.float32).at[0, 0].set(lp['c_b2'][0])
        m, cwu = _edge_mlp(
            ga, gb, dxp, btc,
            lp['e_w2'].T, lp['c_w1'].T, _pad8(lp['c_w2'][0][None, :]),
            _pad8(w_d[None, :]), btab, _pad8(lp['e_b2'][None, :]),
            _pad8(lp['c_b1'][None, :]), misc)

        last = li == len(p['layers']) - 1
        if use_sc:
            parts = _sc_scatter(m, col_i, n)
            m_agg = parts[:n] + parts[n:]
            if not last:
                partsx = _sc_scatter(cwu, col_i, n)
                x = x + partsx[:n, :3] + partsx[n:, :3]
        else:
            m_agg = jnp.zeros_like(h).at[col].add(m)
            if not last:
                x = x + jnp.zeros_like(x).at[col].add(cwu[:, :3])
        h = _node_update(
            h, m_agg, lp['n_w1'][:, :HID].T, lp['n_w1'][:, HID:].T,
            lp['n_w2'].T, _pad8(lp['n_b1']), _pad8(lp['n_b2']),
            _pad8(lp['ln_g']), _pad8(lp['ln_b']))

    w2tp = jnp.zeros((HID, 8), jnp.float32).at[:, :3].set(p['np_w2'].T)
    b2p = jnp.zeros((8, 8), jnp.float32).at[0, :3].set(p['np_b2'])
    return _head(h, p['np_w1'].T, _pad8(p['np_b1']), w2tp, b2p)


# speedup vs baseline: 1.5773x; 1.0201x over previous
import functools
import math

import jax
import jax.numpy as jnp
from jax import lax
from jax.experimental import pallas as pl
from jax.experimental.pallas import tpu as pltpu
from jax.experimental.pallas import tpu_sc as plsc

HID = 128
TIME_DIM = 128
NUM_LAYERS = 6
T_STEPS = 1000


def _schedule():
    s = 0.008
    steps = T_STEPS + 1
    x = jnp.linspace(0.0, float(T_STEPS), steps)
    ac = jnp.cos((x / T_STEPS + s) / (1 + s) * math.pi * 0.5) ** 2
    ac = ac / ac[0]
    betas = jnp.clip(1 - ac[1:] / ac[:-1], 0.0001, 0.9999)
    alphas = 1 - betas
    acp = jnp.cumprod(alphas)
    return jnp.sqrt(acp), jnp.sqrt(1 - acp)


def _sin_emb(t, dim):
    half = dim // 2
    emb = jnp.exp(jnp.arange(half, dtype=jnp.float32) * -(math.log(10000.0) / (half - 1)))
    emb = t[:, None] * emb[None, :]
    return jnp.concatenate([jnp.sin(emb), jnp.cos(emb)], axis=-1)


def _silu(v):
    return v * jax.nn.sigmoid(v)


def _edge_block_kernel(g_ref, dx_ref, bt_ref, w2t_ref, cw1t_ref,
                       cw2p_ref, wd_ref, btab_ref, eb2_ref, cb1_ref, misc_ref,
                       m_ref, cwu_ref):
    g = g_ref[...]
    dx = dx_ref[...]
    d3 = dx[:, 0:3]
    d2 = jnp.sum(d3 * d3, axis=1, keepdims=True)
    dist = jnp.maximum(jnp.sqrt(d2), 1e-6)
    unit = d3 / dist
    bt = bt_ref[...]
    pre = g + dist * wd_ref[0:1, :]
    for k in range(5):
        pre += jnp.where(bt == k, btab_ref[k:k + 1, :], 0.0)
    s1 = _silu(pre)
    m = _silu(jnp.dot(s1, w2t_ref[...], preferred_element_type=jnp.float32)
              + eb2_ref[0:1, :])
    c1 = _silu(jnp.dot(m, cw1t_ref[...], preferred_element_type=jnp.float32)
               + cb1_ref[0:1, :])
    cw = jnp.sum(c1 * cw2p_ref[0:1, :], axis=1, keepdims=True) + misc_ref[0:1, 0:1]
    m_ref[...] = m
    cwu_ref[...] = jnp.concatenate(
        [cw * unit, jnp.zeros((unit.shape[0], 13), jnp.float32)], axis=1)


def _node_block_kernel(h_ref, magg_ref, w1at_ref, w1bt_ref, w2t_ref,
                       nb1_ref, nb2_ref, lng_ref, lnb_ref, out_ref):
    h = h_ref[...]
    magg = magg_ref[...]
    z = (jnp.dot(h, w1at_ref[...], preferred_element_type=jnp.float32)
         + jnp.dot(magg, w1bt_ref[...], preferred_element_type=jnp.float32)
         + nb1_ref[0:1, :])
    hn = jnp.dot(_silu(z), w2t_ref[...],
                 preferred_element_type=jnp.float32) + nb2_ref[0:1, :]
    hr = h + hn
    mu = jnp.mean(hr, axis=1, keepdims=True)
    var = jnp.mean((hr - mu) * (hr - mu), axis=1, keepdims=True)
    out_ref[...] = ((hr - mu) * lax.rsqrt(var + 1e-05) * lng_ref[0:1, :]
                    + lnb_ref[0:1, :])


def _prep_block_kernel(h_ref, whrt_ref, whct_ref, eb1_ref, ta_ref, tb_ref):
    h = h_ref[...]
    ta_ref[...] = jnp.dot(h, whrt_ref[...],
                          preferred_element_type=jnp.float32) + eb1_ref[0:1, :]
    tb_ref[...] = jnp.dot(h, whct_ref[...],
                          preferred_element_type=jnp.float32)


def _head_block_kernel(h_ref, w1t_ref, b1_ref, w2tp_ref, b2_ref, out_ref):
    z = _silu(jnp.dot(h_ref[...], w1t_ref[...],
                      preferred_element_type=jnp.float32) + b1_ref[0:1, :])
    out_ref[...] = jnp.dot(z, w2tp_ref[...],
                           preferred_element_type=jnp.float32) + b2_ref[0:1, :]


def _node_blk(n):
    for b in (2000, 1000, 500, 250, 200, 100, 50, 25, 8):
        if n % b == 0 and b % 8 == 0:
            return b
    return None


def _node_update(h, magg, w1at, w1bt, w2t, nb1, nb2, lng, lnb):
    n = h.shape[0]
    blk = _node_blk(n)
    if blk is None:
        z = h @ w1at + magg @ w1bt + nb1[0:1]
        hr = h + (_silu(z) @ w2t + nb2[0:1])
        mu = hr.mean(-1, keepdims=True)
        var = hr.var(-1, keepdims=True)
        return (hr - mu) / jnp.sqrt(var + 1e-05) * lng[0:1] + lnb[0:1]
    data = pl.BlockSpec((blk, HID), lambda i: (i, 0))
    const = lambda s: pl.BlockSpec(s, lambda i: (0, 0))
    return pl.pallas_call(
        _node_block_kernel,
        grid=(n // blk,),
        in_specs=[data, data, const((HID, HID)), const((HID, HID)),
                  const((HID, HID)), const((8, HID)), const((8, HID)),
                  const((8, HID)), const((8, HID))],
        out_specs=data,
        out_shape=jax.ShapeDtypeStruct((n, HID), jnp.float32),
    )(h, magg, w1at, w1bt, w2t, nb1, nb2, lng, lnb)


def _prep_tables(h, whrt, whct, eb1):
    n = h.shape[0]
    blk = _node_blk(n)
    if blk is None:
        return h @ whrt + eb1[0:1], h @ whct
    data = pl.BlockSpec((blk, HID), lambda i: (i, 0))
    const = lambda s: pl.BlockSpec(s, lambda i: (0, 0))
    return pl.pallas_call(
        _prep_block_kernel,
        grid=(n // blk,),
        in_specs=[data, const((HID, HID)), const((HID, HID)), const((8, HID))],
        out_specs=[data, data],
        out_shape=[jax.ShapeDtypeStruct((n, HID), jnp.float32),
                   jax.ShapeDtypeStruct((n, HID), jnp.float32)],
    )(h, whrt, whct, eb1)


def _head(h, w1t, b1, w2tp, b2p):
    n = h.shape[0]
    blk = _node_blk(n)
    if blk is None:
        return (_silu(h @ w1t + b1[0:1]) @ w2tp + b2p[0:1])[:, :3]
    data = pl.BlockSpec((blk, HID), lambda i: (i, 0))
    const = lambda s: pl.BlockSpec(s, lambda i: (0, 0))
    out = pl.pallas_call(
        _head_block_kernel,
        grid=(n // blk,),
        in_specs=[data, const((HID, HID)), const((8, HID)),
                  const((HID, 8)), const((8, 8))],
        out_specs=pl.BlockSpec((blk, 8), lambda i: (i, 0)),
        out_shape=jax.ShapeDtypeStruct((n, 8), jnp.float32),
    )(h, w1t, b1, w2tp, b2p)
    return out[:, :3]


def _pick_blk(e):
    for b in (1280, 640, 512, 256, 128, 64, 32, 16, 8):
        if e % b == 0:
            return b
    return None


def _edge_mlp(g, dxp, btc, w2t, cw1t, cw2p, wd, btab, eb2, cb1, misc):
    e = g.shape[0]
    blk = _pick_blk(e)
    if blk is None:
        blk = 1280
        epad = ((e + blk - 1) // blk) * blk
        g = jnp.pad(g, ((0, epad - e), (0, 0)))
        dxp = jnp.pad(dxp, ((0, epad - e), (0, 0)))
        btc = jnp.pad(btc, ((0, epad - e), (0, 0)))
    epad = g.shape[0]
    grid = epad // blk
    data_spec = lambda w: pl.BlockSpec((blk, w), lambda i: (i, 0))
    const_spec = lambda s: pl.BlockSpec(s, lambda i: (0, 0))
    m, cwu = pl.pallas_call(
        _edge_block_kernel,
        grid=(grid,),
        in_specs=[
            data_spec(HID), data_spec(16), data_spec(1),
            const_spec((HID, HID)), const_spec((HID, 64)), const_spec((8, 64)),
            const_spec((8, HID)), const_spec((8, HID)), const_spec((8, HID)),
            const_spec((8, 64)), const_spec((8, HID)),
        ],
        out_specs=[
            pl.BlockSpec((blk, HID), lambda i: (i, 0)),
            pl.BlockSpec((blk, 16), lambda i: (i, 0)),
        ],
        out_shape=[
            jax.ShapeDtypeStruct((epad, HID), jnp.float32),
            jax.ShapeDtypeStruct((epad, 16), jnp.float32),
        ],
    )(g, dxp, btc, w2t, cw1t, cw2p, wd, btab, eb2, cb1, misc)
    return m[:e], cwu[:e]


_NC, _NS = 2, 16
_NW = _NC * _NS


def _gather_chunk(epw):
    for k in range(128, 7, -8):
        if epw % k == 0:
            return k
    return None


@functools.lru_cache(maxsize=None)
def _sc_gather_kernel(n, w, e):
    epw = e // _NW
    k_chunk = _gather_chunk(epw)
    n_chunks = epw // k_chunk
    grp = 1
    for g in (5, 4, 3, 2):
        if n_chunks % g == 0:
            grp = g
            break
    mesh = plsc.VectorSubcoreMesh(core_axis_name="c", subcore_axis_name="s")

    @functools.partial(
        pl.kernel, mesh=mesh,
        out_type=jax.ShapeDtypeStruct((e, w), jnp.float32),
        scratch_types=[
            pltpu.VMEM((grp, k_chunk), jnp.int32),
            pltpu.VMEM((grp, k_chunk), jnp.int32),
            pltpu.VMEM((grp, k_chunk, w), jnp.float32),
            pltpu.VMEM((grp, k_chunk, w), jnp.float32),
            pltpu.SemaphoreType.DMA,
            pltpu.SemaphoreType.DMA,
        ],
    )
    def gather_kernel(ta_h, tb_h, row_h, col_h, g_h,
                      idx_r, idx_c, buf_a, buf_b, sem_g, sem_w):
        wid = lax.axis_index("s") * _NC + lax.axis_index("c")
        base = wid * epw
        nsl = w // 16

        def body(j, carry):
            goff = base + j * (grp * k_chunk)
            for b in range(grp):
                off = goff + b * k_chunk
                pltpu.sync_copy(row_h.at[pl.ds(off, k_chunk)], idx_r.at[b])
                pltpu.sync_copy(col_h.at[pl.ds(off, k_chunk)], idx_c.at[b])
            cps = []
            for b in range(grp):
                cps.append(pltpu.async_copy(ta_h.at[idx_r.at[b]],
                                            buf_a.at[b], sem_g))
                cps.append(pltpu.async_copy(tb_h.at[idx_c.at[b]],
                                            buf_b.at[b], sem_g))
            for cp in cps:
                cp.wait()
            wps = []
            for b in range(grp):
                def add_body(i, c2, b=b):
                    for c in range(nsl):
                        sl = pl.ds(c * 16, 16)
                        buf_a[b, i, sl] = buf_a[b, i, sl] + buf_b[b, i, sl]
                    return c2

                lax.fori_loop(0, k_chunk, add_body, 0)
                off = goff + b * k_chunk
                wps.append(pltpu.async_copy(
                    buf_a.at[b], g_h.at[pl.ds(off, k_chunk)], sem_w))
            for wp in wps:
                wp.wait()
            return carry

        lax.fori_loop(0, n_chunks // grp, body, 0)

    return gather_kernel


def _sc_gather(ta, tb, row, col):
    return _sc_gather_kernel(ta.shape[0], ta.shape[1], row.shape[0])(
        ta, tb, row, col)


@functools.lru_cache(maxsize=None)
def _sc_scatter_kernel(e, n):
    epw = e // _NW
    k_chunk = _scatter_chunk(epw, n)
    n_chunks = n // k_chunk
    e_chunks = epw // k_chunk
    grp = 1
    for g in (5, 4, 3, 2):
        if e_chunks % g == 0:
            grp = g
            break
    mesh = plsc.VectorSubcoreMesh(core_axis_name="c", subcore_axis_name="s")
    stripe_iters = (n_chunks + _NS - 1) // _NS

    @functools.partial(
        pl.kernel, mesh=mesh,
        out_type=jax.ShapeDtypeStruct((2 * n, HID), jnp.float32),
        scratch_types=[
            pltpu.VMEM((grp, k_chunk), jnp.int32),
            pltpu.VMEM((grp, k_chunk, HID), jnp.float32),
            pltpu.VMEM((k_chunk, HID), jnp.float32),
            pltpu.VMEM_SHARED((n, HID), jnp.float32),
            pltpu.SemaphoreType.DMA,
        ],
    )
    def scatter_kernel(m_h, col_h, z_h, out_h, idx_v, m_v, z_v, acc_sh, sem_l):
        cid = lax.axis_index("c")
        sid = lax.axis_index("s")
        wid = sid * _NC + cid
        base = wid * epw

        pltpu.sync_copy(z_h, z_v)

        def zero_body(j, carry):
            chunk = j * _NS + sid

            @pl.when(chunk < n_chunks)
            def _():
                pltpu.sync_copy(z_v, acc_sh.at[pl.ds(chunk * k_chunk, k_chunk)])
            return carry

        lax.fori_loop(0, stripe_iters, zero_body, 0)
        plsc.subcore_barrier()

        def body(j, carry):
            goff = base + j * (grp * k_chunk)
            cps = []
            for b in range(grp):
                off = goff + b * k_chunk
                pltpu.sync_copy(col_h.at[pl.ds(off, k_chunk)], idx_v.at[b])
                cps.append(pltpu.async_copy(m_h.at[pl.ds(off, k_chunk)],
                                            m_v.at[b], sem_l))
            for cp in cps:
                cp.wait()
            for b in range(grp):
                pltpu.sync_copy(m_v.at[b], acc_sh.at[idx_v.at[b]], add=True)
            return carry

        lax.fori_loop(0, e_chunks // grp, body, 0)
        plsc.subcore_barrier()

        def out_body(j, carry):
            chunk = j * _NS + sid

            @pl.when(chunk < n_chunks)
            def _():
                pltpu.sync_copy(acc_sh.at[pl.ds(chunk * k_chunk, k_chunk)],
                                out_h.at[pl.ds(cid * n + chunk * k_chunk, k_chunk)])
            return carry

        lax.fori_loop(0, stripe_iters, out_body, 0)

    return scatter_kernel


@functools.lru_cache(maxsize=None)
def _sc_scatter16_kernel(e, n):
    epw = e // _NW
    k_chunk = _scatter_chunk(epw, n)
    n_chunks = n // k_chunk
    e_chunks = epw // k_chunk
    grp = 1
    for g in (5, 4, 3, 2):
        if e_chunks % g == 0:
            grp = g
            break
    mesh = plsc.VectorSubcoreMesh(core_axis_name="c", subcore_axis_name="s")
    stripe_iters = (n_chunks + _NS - 1) // _NS

    @functools.partial(
        pl.kernel, mesh=mesh,
        out_type=jax.ShapeDtypeStruct((2 * n, 16), jnp.float32),
        scratch_types=[
            pltpu.VMEM((grp, k_chunk), jnp.int32),
            pltpu.VMEM((grp, k_chunk, 16), jnp.float32),
            pltpu.VMEM((k_chunk, 16), jnp.float32),
            pltpu.VMEM_SHARED((n, 16), jnp.float32),
            pltpu.SemaphoreType.DMA,
        ],
    )
    def scatter_kernel(c_h, col_h, z_h, out_h, idx_v, c_v, z_v, acc_sh, sem_l):
        cid = lax.axis_index("c")
        sid = lax.axis_index("s")
        wid = sid * _NC + cid
        base = wid * epw

        pltpu.sync_copy(z_h, z_v)

        def zero_body(j, carry):
            chunk = j * _NS + sid

            @pl.when(chunk < n_chunks)
            def _():
                pltpu.sync_copy(z_v, acc_sh.at[pl.ds(chunk * k_chunk, k_chunk)])
            return carry

        lax.fori_loop(0, stripe_iters, zero_body, 0)
        plsc.subcore_barrier()

        def body(j, carry):
            goff = base + j * (grp * k_chunk)
            cps = []
            for b in range(grp):
                off = goff + b * k_chunk
                pltpu.sync_copy(col_h.at[pl.ds(off, k_chunk)], idx_v.at[b])
                cps.append(pltpu.async_copy(c_h.at[pl.ds(off, k_chunk)],
                                            c_v.at[b], sem_l))
            for cp in cps:
                cp.wait()
            for b in range(grp):
                pltpu.sync_copy(c_v.at[b], acc_sh.at[idx_v.at[b]], add=True)
            return carry

        lax.fori_loop(0, e_chunks // grp, body, 0)
        plsc.subcore_barrier()

        def out_body(j, carry):
            chunk = j * _NS + sid

            @pl.when(chunk < n_chunks)
            def _():
                pltpu.sync_copy(acc_sh.at[pl.ds(chunk * k_chunk, k_chunk)],
                                out_h.at[pl.ds(cid * n + chunk * k_chunk, k_chunk)])
            return carry

        lax.fori_loop(0, stripe_iters, out_body, 0)

    return scatter_kernel


def _sc_scatter16(cwu, col, n):
    e = cwu.shape[0]
    k_chunk = _scatter_chunk(e // _NW, n)
    zeros = jnp.zeros((k_chunk, 16), jnp.float32)
    return _sc_scatter16_kernel(e, n)(cwu, col, zeros)


def _scatter_chunk(epw, n):
    for k in range(64, 7, -8):
        if epw % k == 0 and n % k == 0:
            return k
    return None


def _sc_scatter(m, col, n):
    e = m.shape[0]
    k_chunk = _scatter_chunk(e // _NW, n)
    zeros = jnp.zeros((k_chunk, HID), jnp.float32)
    return _sc_scatter_kernel(e, n)(m, col, zeros)


def _pad8(v, rows=8):
    v = jnp.reshape(v, (-1,))
    out = jnp.zeros((rows, v.shape[0]), jnp.float32)
    return out.at[0, :].set(v)


def kernel(x_0, t, atom_types, edge_index, bond_types, batch_idx, noise, params):
    p = params
    sqa, sqm = _schedule()
    sa = sqa[t][batch_idx][:, None]
    sm = sqm[t][batch_idx][:, None]
    x_t = sa * x_0 + sm * noise

    h = p['atom_embed'][jnp.clip(atom_types, 0, 9)]
    h = h + x_t @ p['coord_w'].T + p['coord_b']
    te = _sin_emb(t.astype(jnp.float32), TIME_DIM)
    te = _silu(te @ p['time_w1'].T + p['time_b1']) @ p['time_w2'].T + p['time_b2']
    h = h + te[batch_idx]

    row, col = edge_index[0], edge_index[1]
    btc = jnp.clip(bond_types, 0, 4).astype(jnp.int32)[:, None]
    n = x_0.shape[0]
    x = x_t
    e = row.shape[0]
    use_sc = (e % _NW == 0 and _gather_chunk(e // _NW) is not None
              and _scatter_chunk(e // _NW, n) is not None)
    row_i = row.astype(jnp.int32)
    col_i = col.astype(jnp.int32)

    for li, lp in enumerate(p['layers']):
        w_hr = lp['e_w1'][:, :HID]
        w_hc = lp['e_w1'][:, HID:2 * HID]
        w_ea = lp['e_w1'][:, 2 * HID:2 * HID + 32]
        w_d = lp['e_w1'][:, 2 * HID + 32]
        ta, tb = _prep_tables(h, w_hr.T, w_hc.T, _pad8(lp['e_b1']))
        if use_sc:
            g = _sc_gather(ta, tb, row_i, col_i)
        else:
            g = jnp.take(ta, row, axis=0) + jnp.take(tb, col, axis=0)
        dx3 = jnp.take(x, row, axis=0) - jnp.take(x, col, axis=0)
        dxp = jnp.pad(dx3, ((0, 0), (0, 13)))

        btab6 = p['bond_embed'] @ w_ea.T
        btab = jnp.zeros((8, HID), jnp.float32).at[:6].set(btab6)
        misc = jnp.zeros((8, HID), jnp.float32).at[0, 0].set(lp['c_b2'][0])
        m, cwu = _edge_mlp(
            g, dxp, btc,
            lp['e_w2'].T, lp['c_w1'].T, _pad8(lp['c_w2'][0][None, :]),
            _pad8(w_d[None, :]), btab, _pad8(lp['e_b2'][None, :]),
            _pad8(lp['c_b1'][None, :]), misc)

        last = li == len(p['layers']) - 1
        if use_sc:
            parts = _sc_scatter(m, col_i, n)
            m_agg = parts[:n] + parts[n:]
            if not last:
                partsx = _sc_scatter16(cwu, col_i, n)
                x = x + partsx[:n, :3] + partsx[n:, :3]
        else:
            m_agg = jnp.zeros_like(h).at[col].add(m)
            if not last:
                x = x + jnp.zeros_like(x).at[col].add(cwu[:, :3])
        h = _node_update(
            h, m_agg, lp['n_w1'][:, :HID].T, lp['n_w1'][:, HID:].T,
            lp['n_w2'].T, _pad8(lp['n_b1']), _pad8(lp['n_b2']),
            _pad8(lp['ln_g']), _pad8(lp['ln_b']))

    w2tp = jnp.zeros((HID, 8), jnp.float32).at[:, :3].set(p['np_w2'].T)
    b2p = jnp.zeros((8, 8), jnp.float32).at[0, :3].set(p['np_b2'])
    return _head(h, p['np_w1'].T, _pad8(p['np_b1']), w2tp, b2p)
